# baseline (speedup 1.0000x reference)
import functools

import jax
import jax.numpy as jnp
from jax import lax
from jax.experimental import pallas as pl
from jax.experimental.pallas import tpu as pltpu
from jax.experimental.pallas import tpu_sc as plsc

B = 16384
E = 32
L = 16

_info = plsc.get_sparse_core_info()
_NC, _NS = _info.num_cores, _info.num_subcores
NW = _NC * _NS
BPW = B // NW


PITCH = 17


def _sc_kernel(user_hbm, item_hbm, ut_hbm, it_hbm, out_hbm,
               uidx_v, iidx_v, urow_v, irow_v, out_v, t_v, sem_u, sem_i):
    wid = lax.axis_index("s") * _NC + lax.axis_index("c")
    base = wid * BPW
    pltpu.sync_copy(user_hbm.at[pl.ds(base, BPW)], uidx_v)
    pltpu.sync_copy(item_hbm.at[pl.ds(base, BPW)], iidx_v)
    cp_u = pltpu.async_copy(ut_hbm.at[uidx_v], urow_v, sem_u)
    cp_i = pltpu.async_copy(it_hbm.at[iidx_v], irow_v, sem_i)
    cp_u.wait()
    cp_i.wait()

    lanes = lax.iota(jnp.int32, L)

    def body(g, carry):
        for j in range(L):
            i = g * L + j
            u1 = urow_v[i, pl.ds(0, L)]
            u2 = urow_v[i, pl.ds(L, L)]
            v1 = irow_v[i, pl.ds(0, L)]
            v2 = irow_v[i, pl.ds(L, L)]
            s = u1 * v1 + u2 * v2
            plsc.store_scatter(t_v, [lanes * PITCH + j], s)
        acc = t_v[pl.ds(0, L)]
        for l in range(1, L):
            acc = acc + t_v[pl.ds(l * PITCH, L)]
        out_v[pl.ds(g * L, L)] = acc
        return carry

    lax.fori_loop(0, BPW // L, body, 0)
    pltpu.sync_copy(out_v, out_hbm.at[pl.ds(base, BPW)])


@jax.jit
def kernel(user, item, user_table, item_table):
    user = user.astype(jnp.int32)
    item = item.astype(jnp.int32)
    mesh = plsc.VectorSubcoreMesh(core_axis_name="c", subcore_axis_name="s")
    f = functools.partial(
        pl.kernel,
        mesh=mesh,
        out_type=jax.ShapeDtypeStruct((B,), jnp.float32),
        compiler_params=pltpu.CompilerParams(
            needs_layout_passes=False, use_tc_tiling_on_sc=False),
        scratch_types=[
            pltpu.VMEM((BPW,), jnp.int32),
            pltpu.VMEM((BPW,), jnp.int32),
            pltpu.VMEM((BPW, E), jnp.float32),
            pltpu.VMEM((BPW, E), jnp.float32),
            pltpu.VMEM((BPW,), jnp.float32),
            pltpu.VMEM((L * PITCH,), jnp.float32),
            pltpu.SemaphoreType.DMA,
            pltpu.SemaphoreType.DMA,
        ],
    )(_sc_kernel)
    return f(user, item, user_table, item_table)


# speedup vs baseline: 1.4975x; 1.4975x over previous
import functools

import jax
import jax.numpy as jnp
from jax import lax
from jax.experimental import pallas as pl
from jax.experimental.pallas import tpu as pltpu
from jax.experimental.pallas import tpu_sc as plsc

B = 16384
E = 32
L = 16
PITCH = 17

_info = plsc.get_sparse_core_info()
_NC, _NS = _info.num_cores, _info.num_subcores
NW = _NC * _NS
BPW = B // NW
CHUNK = 256
NCHUNK = BPW // CHUNK


def _sc_kernel(user_hbm, item_hbm, ut_hbm, it_hbm, out_hbm,
               uidx_v, iidx_v, urow_v, irow_v, out_v, t_v, sem_u, sem_i):
    wid = lax.axis_index("s") * _NC + lax.axis_index("c")
    base = wid * BPW
    pltpu.sync_copy(user_hbm.at[pl.ds(base, BPW)], uidx_v)
    pltpu.sync_copy(item_hbm.at[pl.ds(base, BPW)], iidx_v)
    lanes = lax.iota(jnp.int32, L)

    for c in range(NCHUNK):
        c0 = c * CHUNK

        def fire(g, carry):
            uvec = uidx_v[pl.ds(c0 + g * L, L)]
            ivec = iidx_v[pl.ds(c0 + g * L, L)]
            for j in range(L):
                i = g * L + j
                pltpu.make_async_copy(
                    ut_hbm.at[pl.ds(uvec[j], 1), :],
                    urow_v.at[pl.ds(i, 1), :], sem_u).start()
                pltpu.make_async_copy(
                    it_hbm.at[pl.ds(ivec[j], 1), :],
                    irow_v.at[pl.ds(i, 1), :], sem_i).start()
            return carry

        lax.fori_loop(0, CHUNK // L, fire, 0)
        pltpu.make_async_copy(ut_hbm.at[pl.ds(0, CHUNK), :], urow_v,
                              sem_u).wait()
        pltpu.make_async_copy(it_hbm.at[pl.ds(0, CHUNK), :], irow_v,
                              sem_i).wait()

        def compute(g, carry):
            for j in range(L):
                i = g * L + j
                u1 = urow_v[i, pl.ds(0, L)]
                u2 = urow_v[i, pl.ds(L, L)]
                v1 = irow_v[i, pl.ds(0, L)]
                v2 = irow_v[i, pl.ds(L, L)]
                s = u1 * v1 + u2 * v2
                plsc.store_scatter(t_v, [lanes * PITCH + j], s)
            acc = t_v[pl.ds(0, L)]
            for l in range(1, L):
                acc = acc + t_v[pl.ds(l * PITCH, L)]
            out_v[pl.ds(c0 + g * L, L)] = acc
            return carry

        lax.fori_loop(0, CHUNK // L, compute, 0)

    pltpu.sync_copy(out_v, out_hbm.at[pl.ds(base, BPW)])


@jax.jit
def kernel(user, item, user_table, item_table):
    user = user.astype(jnp.int32)
    item = item.astype(jnp.int32)
    mesh = plsc.VectorSubcoreMesh(core_axis_name="c", subcore_axis_name="s")
    f = functools.partial(
        pl.kernel,
        mesh=mesh,
        out_type=jax.ShapeDtypeStruct((B,), jnp.float32),
        compiler_params=pltpu.CompilerParams(needs_layout_passes=False),
        scratch_types=[
            pltpu.VMEM((BPW,), jnp.int32),
            pltpu.VMEM((BPW,), jnp.int32),
            pltpu.VMEM((CHUNK, E), jnp.float32),
            pltpu.VMEM((CHUNK, E), jnp.float32),
            pltpu.VMEM((BPW,), jnp.float32),
            pltpu.VMEM((L * PITCH,), jnp.float32),
            pltpu.SemaphoreType.DMA,
            pltpu.SemaphoreType.DMA,
        ],
    )(_sc_kernel)
    return f(user, item, user_table, item_table)
